# baseline (speedup 1.0000x reference)
import functools

import jax
import jax.numpy as jnp
from jax import lax
from jax.experimental import pallas as pl
from jax.experimental.pallas import tpu as pltpu
from jax.experimental.pallas import tpu_sc as plsc

L = 16
NC, NS = 2, 16
NW = NC * NS
B = 1024 * 1024
PT = B // NW
C = 512
NCH = PT // C
NG = C // L
IDXW = 128
NB = C // IDXW
DIMS = (4096, 2048, 1024, 512)
NTAP = 16

_mesh = plsc.VectorSubcoreMesh(core_axis_name="c", subcore_axis_name="s")


@functools.partial(
    pl.kernel,
    mesh=_mesh,
    out_type=jax.ShapeDtypeStruct((B,), jnp.float32),
    scratch_types=[
        pltpu.VMEM((C,), jnp.float32),
        pltpu.VMEM((C,), jnp.float32),
        pltpu.VMEM((C,), jnp.float32),
        pltpu.VMEM((C,), jnp.float32),
        pltpu.VMEM((NTAP * C,), jnp.int32),
        pltpu.VMEM((NTAP * C,), jnp.float32),
        pltpu.VMEM((NTAP * C,), jnp.int32),
        pltpu.VMEM((NTAP * C,), jnp.float32),
        pltpu.VMEM((NTAP * C,), jnp.float32),
        pltpu.VMEM((NTAP * C,), jnp.float32),
        pltpu.VMEM((C,), jnp.float32),
        pltpu.VMEM((C,), jnp.float32),
        pltpu.SemaphoreType.DMA,
        pltpu.SemaphoreType.DMA,
        pltpu.SemaphoreType.DMA,
    ],
)
def _tex_sc(u_hbm, v_hbm, t1, t2, t3, t4, out_hbm,
            u0_v, v0_v, u1_v, v1_v, idx0_v, w0_v, idx1_v, w1_v,
            val0_v, val1_v, out0_v, out1_v, sem, sem_uv, sem_out):
    wid = lax.axis_index("s") * NC + lax.axis_index("c")
    tabs = (t1, t2, t3, t4)

    def uv_load(ci, u_v, v_v):
        base = wid * PT + ci * C
        return (
            pltpu.async_copy(u_hbm.at[pl.ds(base, C)], u_v, sem_uv),
            pltpu.async_copy(v_hbm.at[pl.ds(base, C)], v_v, sem_uv),
        )

    def compute_chunk(u_v, v_v, idx_v, w_v):
        def grp(g, c2):
            off = g * L
            uu = u_v[pl.ds(off, L)]
            vv = v_v[pl.ds(off, L)]
            for lvl in range(4):
                D = DIMS[lvl]
                fD = float(D)
                ix = uu * fD - 0.5
                iy = vv * fD - 0.5
                xt = ix.astype(jnp.int32)
                yt = iy.astype(jnp.int32)
                x0 = jnp.where(ix < xt.astype(jnp.float32), xt - 1, xt)
                y0 = jnp.where(iy < yt.astype(jnp.float32), yt - 1, yt)
                fx = ix - x0.astype(jnp.float32)
                fy = iy - y0.astype(jnp.float32)
                x1 = x0 + 1
                y1 = y0 + 1
                vx0 = (x0 >= 0) & (x0 <= D - 1)
                vx1 = (x1 >= 0) & (x1 <= D - 1)
                vy0 = (y0 >= 0) & (y0 <= D - 1)
                vy1 = (y1 >= 0) & (y1 <= D - 1)
                xc0 = jnp.clip(x0, 0, D - 1)
                xc1 = jnp.clip(x1, 0, D - 1)
                yb0 = jnp.clip(y0, 0, D - 1) * D
                yb1 = jnp.clip(y1, 0, D - 1) * D
                wx0 = 1.0 - fx
                wy0 = 1.0 - fy
                taps = (
                    (yb0 + xc0, wy0 * wx0, vy0 & vx0),
                    (yb0 + xc1, wy0 * fx, vy0 & vx1),
                    (yb1 + xc0, fy * wx0, vy1 & vx0),
                    (yb1 + xc1, fy * fx, vy1 & vx1),
                )
                for t, (fi, ww, vld) in enumerate(taps):
                    r = lvl * 4 + t
                    idx_v[pl.ds(r * C + off, L)] = fi
                    w_v[pl.ds(r * C + off, L)] = jnp.where(vld, ww, 0.0)
            return c2

        lax.fori_loop(0, NG, grp, 0)

    def fire(idx_v, val_v):
        return [
            pltpu.async_copy(
                tabs[l].at[idx_v.at[pl.ds(l * 4 * C, 4 * C)]],
                val_v.at[pl.ds(l * 4 * C, 4 * C)],
                sem,
            )
            for l in range(4)
        ]

    def drain_acc(ci, handles, val_v, w_v, out_v):
        for h in handles:
            h.wait()

        def acc(g, c2):
            off = g * L
            a = None
            for r in range(NTAP):
                term = (val_v[pl.ds(r * C + off, L)]
                        * w_v[pl.ds(r * C + off, L)])
                a = term if a is None else a + term
            out_v[pl.ds(off, L)] = a
            return c2

        lax.fori_loop(0, NG, acc, 0)
        return pltpu.async_copy(
            out_v, out_hbm.at[pl.ds(wid * PT + ci * C, C)], sem_out)

    for h in uv_load(0, u0_v, v0_v):
        h.wait()
    compute_chunk(u0_v, v0_v, idx0_v, w0_v)

    def pair_body(i, carry):
        c0 = 2 * i
        c1 = c0 + 1
        cn = jnp.where(c1 + 1 < NCH, c1 + 1, 0)
        hu1 = uv_load(c1, u1_v, v1_v)
        h0 = fire(idx0_v, val0_v)
        for h in hu1:
            h.wait()
        compute_chunk(u1_v, v1_v, idx1_v, w1_v)
        hu2 = uv_load(cn, u0_v, v0_v)
        h1 = fire(idx1_v, val1_v)
        ho0 = drain_acc(c0, h0, val0_v, w0_v, out0_v)
        for h in hu2:
            h.wait()
        compute_chunk(u0_v, v0_v, idx0_v, w0_v)
        ho1 = drain_acc(c1, h1, val1_v, w1_v, out1_v)
        ho0.wait()
        ho1.wait()
        return carry

    lax.fori_loop(0, NCH // 2, pair_body, 0)


def kernel(x, layer1, layer2, layer3, layer4):
    u = x[0, :, :, 0].reshape(-1)
    v = x[0, :, :, 1].reshape(-1)
    y = _tex_sc(u, v,
                layer1.reshape(-1), layer2.reshape(-1),
                layer3.reshape(-1), layer4.reshape(-1))
    return y.reshape(1, 1, 1024, 1024)


# speedup vs baseline: 1.0165x; 1.0165x over previous
import functools

import jax
import jax.numpy as jnp
from jax import lax
from jax.experimental import pallas as pl
from jax.experimental.pallas import tpu as pltpu
from jax.experimental.pallas import tpu_sc as plsc

L = 16
NC, NS = 2, 16
NW = NC * NS
B = 1024 * 1024
PT = B // NW
C = 1024
NCH = PT // C
NG = C // L
DIMS = (4096, 2048, 1024, 512)
NTAP = 16

_mesh = plsc.VectorSubcoreMesh(core_axis_name="c", subcore_axis_name="s")


@functools.partial(
    pl.kernel,
    mesh=_mesh,
    out_type=jax.ShapeDtypeStruct((B,), jnp.float32),
    scratch_types=[
        pltpu.VMEM((C,), jnp.float32),
        pltpu.VMEM((C,), jnp.float32),
        pltpu.VMEM((C,), jnp.float32),
        pltpu.VMEM((C,), jnp.float32),
        pltpu.VMEM((NTAP * C,), jnp.int32),
        pltpu.VMEM((NTAP * C,), jnp.float32),
        pltpu.VMEM((NTAP * C,), jnp.int32),
        pltpu.VMEM((NTAP * C,), jnp.float32),
        pltpu.VMEM((NTAP * C,), jnp.float32),
        pltpu.VMEM((NTAP * C,), jnp.float32),
        pltpu.VMEM((C,), jnp.float32),
        pltpu.VMEM((C,), jnp.float32),
        pltpu.SemaphoreType.DMA,
        pltpu.SemaphoreType.DMA,
        pltpu.SemaphoreType.DMA,
    ],
)
def _tex_sc(u_hbm, v_hbm, t1, t2, t3, t4, out_hbm,
            u0_v, v0_v, u1_v, v1_v, idx0_v, w0_v, idx1_v, w1_v,
            val0_v, val1_v, out0_v, out1_v, sem, sem_uv, sem_out):
    wid = lax.axis_index("s") * NC + lax.axis_index("c")
    tabs = (t1, t2, t3, t4)

    def uv_load(ci, u_v, v_v):
        base = wid * PT + ci * C
        return (
            pltpu.async_copy(u_hbm.at[pl.ds(base, C)], u_v, sem_uv),
            pltpu.async_copy(v_hbm.at[pl.ds(base, C)], v_v, sem_uv),
        )

    def compute_chunk(u_v, v_v, idx_v, w_v):
        def grp(g, c2):
            off = g * L
            uu = u_v[pl.ds(off, L)]
            vv = v_v[pl.ds(off, L)]
            for lvl in range(4):
                D = DIMS[lvl]
                fD = float(D)
                ix = uu * fD - 0.5
                iy = vv * fD - 0.5
                xt = ix.astype(jnp.int32)
                yt = iy.astype(jnp.int32)
                x0 = jnp.where(ix < xt.astype(jnp.float32), xt - 1, xt)
                y0 = jnp.where(iy < yt.astype(jnp.float32), yt - 1, yt)
                fx = ix - x0.astype(jnp.float32)
                fy = iy - y0.astype(jnp.float32)
                x1 = x0 + 1
                y1 = y0 + 1
                vx0 = (x0 >= 0) & (x0 <= D - 1)
                vx1 = (x1 >= 0) & (x1 <= D - 1)
                vy0 = (y0 >= 0) & (y0 <= D - 1)
                vy1 = (y1 >= 0) & (y1 <= D - 1)
                xc0 = jnp.clip(x0, 0, D - 1)
                xc1 = jnp.clip(x1, 0, D - 1)
                yb0 = jnp.clip(y0, 0, D - 1) * D
                yb1 = jnp.clip(y1, 0, D - 1) * D
                wx0 = 1.0 - fx
                wy0 = 1.0 - fy
                taps = (
                    (yb0 + xc0, wy0 * wx0, vy0 & vx0),
                    (yb0 + xc1, wy0 * fx, vy0 & vx1),
                    (yb1 + xc0, fy * wx0, vy1 & vx0),
                    (yb1 + xc1, fy * fx, vy1 & vx1),
                )
                for t, (fi, ww, vld) in enumerate(taps):
                    r = lvl * 4 + t
                    idx_v[pl.ds(r * C + off, L)] = fi
                    w_v[pl.ds(r * C + off, L)] = jnp.where(vld, ww, 0.0)
            return c2

        lax.fori_loop(0, NG, grp, 0)

    def fire(idx_v, val_v):
        return [
            pltpu.async_copy(
                tabs[l].at[idx_v.at[pl.ds(l * 4 * C, 4 * C)]],
                val_v.at[pl.ds(l * 4 * C, 4 * C)],
                sem,
            )
            for l in range(4)
        ]

    def drain_acc(ci, handles, val_v, w_v, out_v):
        for h in handles:
            h.wait()

        def acc(g, c2):
            off = g * L
            a = None
            for r in range(NTAP):
                term = (val_v[pl.ds(r * C + off, L)]
                        * w_v[pl.ds(r * C + off, L)])
                a = term if a is None else a + term
            out_v[pl.ds(off, L)] = a
            return c2

        lax.fori_loop(0, NG, acc, 0)
        return pltpu.async_copy(
            out_v, out_hbm.at[pl.ds(wid * PT + ci * C, C)], sem_out)

    for h in uv_load(0, u0_v, v0_v):
        h.wait()
    compute_chunk(u0_v, v0_v, idx0_v, w0_v)

    def pair_body(i, carry):
        c0 = 2 * i
        c1 = c0 + 1
        cn = jnp.where(c1 + 1 < NCH, c1 + 1, 0)
        hu1 = uv_load(c1, u1_v, v1_v)
        h0 = fire(idx0_v, val0_v)
        for h in hu1:
            h.wait()
        compute_chunk(u1_v, v1_v, idx1_v, w1_v)
        hu2 = uv_load(cn, u0_v, v0_v)
        h1 = fire(idx1_v, val1_v)
        ho0 = drain_acc(c0, h0, val0_v, w0_v, out0_v)
        for h in hu2:
            h.wait()
        compute_chunk(u0_v, v0_v, idx0_v, w0_v)
        ho1 = drain_acc(c1, h1, val1_v, w1_v, out1_v)
        ho0.wait()
        ho1.wait()
        return carry

    lax.fori_loop(0, NCH // 2, pair_body, 0)


def kernel(x, layer1, layer2, layer3, layer4):
    u = x[0, :, :, 0].reshape(-1)
    v = x[0, :, :, 1].reshape(-1)
    y = _tex_sc(u, v,
                layer1.reshape(-1), layer2.reshape(-1),
                layer3.reshape(-1), layer4.reshape(-1))
    return y.reshape(1, 1, 1024, 1024)
